# baseline (speedup 1.0000x reference)
import jax
import jax.numpy as jnp
from jax.experimental import pallas as pl
from jax.experimental.pallas import tpu as pltpu

_X_SIZE = 512.0
_Y_SIZE = 512.0
_D = 256
_E = 768
_TOK = 2048


def _mlp_block(x_ref, y_ref, w1_ref, b1_ref, gamma_ref, beta_ref, w2_ref,
               b2_ref, out_ref):
    w1 = w1_ref[:]
    a = w1[0]
    b = w1[1]
    c = b1_ref[:]
    abar = jnp.mean(a)
    bbar = jnp.mean(b)
    cbar = jnp.mean(c)
    ac = a - abar
    bc = b - bbar
    cc = c - cbar
    A2 = jnp.mean(ac * ac)
    B2 = jnp.mean(bc * bc)
    C2 = jnp.mean(cc * cc)
    AB = jnp.mean(ac * bc)
    AC = jnp.mean(ac * cc)
    BC = jnp.mean(bc * cc)
    gamma = gamma_ref[:]
    Ag = (ac * gamma)[None, :]
    Bg = (bc * gamma)[None, :]
    Cg = (cc * gamma)[None, :]
    beta = beta_ref[:][None, :]

    xr = x_ref[0, 0, :].astype(jnp.float32)
    yr = y_ref[0, 0, :].astype(jnp.float32)
    xn = xr * (1.0 / _X_SIZE) - 0.5
    yn = yr * (1.0 / _Y_SIZE) - 0.5
    var = (xn * xn * A2 + yn * yn * B2 + C2
           + 2.0 * (xn * yn * AB + xn * AC + yn * BC))
    s = jax.lax.rsqrt(var + 1e-5)
    u = (xn * s).reshape(_TOK, 1)
    v = (yn * s).reshape(_TOK, 1)
    s2 = s.reshape(_TOK, 1)

    hr = jnp.maximum(u * Ag + v * Bg + s2 * Cg + beta, 0.0)
    out = jnp.dot(hr.astype(jnp.bfloat16), w2_ref[:],
                  preferred_element_type=jnp.float32)
    out_ref[:, :] = out + b2_ref[:][None, :]


def kernel(x, y, W1, b1, gamma, beta, W2, b2):
    B, L = x.shape
    n = B * L
    grid = n // _TOK
    x3 = x.reshape(grid, 1, _TOK)
    y3 = y.reshape(grid, 1, _TOK)
    out = pl.pallas_call(
        _mlp_block,
        grid=(grid,),
        in_specs=[
            pl.BlockSpec((1, 1, _TOK), lambda i: (i, 0, 0)),
            pl.BlockSpec((1, 1, _TOK), lambda i: (i, 0, 0)),
            pl.BlockSpec((2, _D), lambda i: (0, 0)),
            pl.BlockSpec((_D,), lambda i: (0,)),
            pl.BlockSpec((_D,), lambda i: (0,)),
            pl.BlockSpec((_D,), lambda i: (0,)),
            pl.BlockSpec((_D, _E), lambda i: (0, 0)),
            pl.BlockSpec((_E,), lambda i: (0,)),
        ],
        out_specs=pl.BlockSpec((_TOK, _E), lambda i: (i, 0)),
        out_shape=jax.ShapeDtypeStruct((n, _E), jnp.float32),
        compiler_params=pltpu.CompilerParams(
            dimension_semantics=("parallel",)),
    )(x3, y3, W1, b1, gamma, beta, W2.astype(jnp.bfloat16), b2)
    return out.reshape(B, L, _E)


# speedup vs baseline: 1.0067x; 1.0067x over previous
import jax
import jax.numpy as jnp
from jax.experimental import pallas as pl
from jax.experimental.pallas import tpu as pltpu

_X_SIZE = 512.0
_Y_SIZE = 512.0
_D = 256
_E = 768
_TOK = 8192


def _mlp_block(x_ref, y_ref, w1_ref, b1_ref, gamma_ref, beta_ref, w2_ref,
               b2_ref, out_ref):
    w1 = w1_ref[:]
    a = w1[0]
    b = w1[1]
    c = b1_ref[:]
    abar = jnp.mean(a)
    bbar = jnp.mean(b)
    cbar = jnp.mean(c)
    ac = a - abar
    bc = b - bbar
    cc = c - cbar
    A2 = jnp.mean(ac * ac)
    B2 = jnp.mean(bc * bc)
    C2 = jnp.mean(cc * cc)
    AB = jnp.mean(ac * bc)
    AC = jnp.mean(ac * cc)
    BC = jnp.mean(bc * cc)
    gamma = gamma_ref[:]
    Ag = (ac * gamma)[None, :]
    Bg = (bc * gamma)[None, :]
    Cg = (cc * gamma)[None, :]
    beta = beta_ref[:][None, :]

    xr = x_ref[0, 0, :].astype(jnp.float32)
    yr = y_ref[0, 0, :].astype(jnp.float32)
    xn = xr * (1.0 / _X_SIZE) - 0.5
    yn = yr * (1.0 / _Y_SIZE) - 0.5
    var = (xn * xn * A2 + yn * yn * B2 + C2
           + 2.0 * (xn * yn * AB + xn * AC + yn * BC))
    s = jax.lax.rsqrt(var + 1e-5)
    u = (xn * s).reshape(_TOK, 1)
    v = (yn * s).reshape(_TOK, 1)
    s2 = s.reshape(_TOK, 1)

    hr = jnp.maximum(u * Ag + v * Bg + s2 * Cg + beta, 0.0)
    out = jnp.dot(hr.astype(jnp.bfloat16), w2_ref[:],
                  preferred_element_type=jnp.float32)
    out_ref[:, :] = out + b2_ref[:][None, :]


def kernel(x, y, W1, b1, gamma, beta, W2, b2):
    B, L = x.shape
    n = B * L
    grid = n // _TOK
    x3 = x.reshape(grid, 1, _TOK)
    y3 = y.reshape(grid, 1, _TOK)
    out = pl.pallas_call(
        _mlp_block,
        grid=(grid,),
        in_specs=[
            pl.BlockSpec((1, 1, _TOK), lambda i: (i, 0, 0)),
            pl.BlockSpec((1, 1, _TOK), lambda i: (i, 0, 0)),
            pl.BlockSpec((2, _D), lambda i: (0, 0)),
            pl.BlockSpec((_D,), lambda i: (0,)),
            pl.BlockSpec((_D,), lambda i: (0,)),
            pl.BlockSpec((_D,), lambda i: (0,)),
            pl.BlockSpec((_D, _E), lambda i: (0, 0)),
            pl.BlockSpec((_E,), lambda i: (0,)),
        ],
        out_specs=pl.BlockSpec((_TOK, _E), lambda i: (i, 0)),
        out_shape=jax.ShapeDtypeStruct((n, _E), jnp.float32),
        compiler_params=pltpu.CompilerParams(
            dimension_semantics=("parallel",)),
    )(x3, y3, W1, b1, gamma, beta, W2.astype(jnp.bfloat16), b2)
    return out.reshape(B, L, _E)


# speedup vs baseline: 1.0575x; 1.0505x over previous
import jax
import jax.numpy as jnp
from jax.experimental import pallas as pl
from jax.experimental.pallas import tpu as pltpu

_X_SIZE = 512.0
_Y_SIZE = 512.0
_D = 256
_E = 768
_TOK = 4096


def _mlp_block(x_ref, y_ref, w1_ref, b1_ref, gamma_ref, beta_ref, w2_ref,
               b2_ref, out_ref):
    w1 = w1_ref[:]
    a = w1[0]
    b = w1[1]
    c = b1_ref[:]
    abar = jnp.mean(a)
    bbar = jnp.mean(b)
    cbar = jnp.mean(c)
    ac = a - abar
    bc = b - bbar
    cc = c - cbar
    A2 = jnp.mean(ac * ac)
    B2 = jnp.mean(bc * bc)
    C2 = jnp.mean(cc * cc)
    AB = jnp.mean(ac * bc)
    AC = jnp.mean(ac * cc)
    BC = jnp.mean(bc * cc)
    gamma = gamma_ref[:]
    Ag = (ac * gamma)[None, :]
    Bg = (bc * gamma)[None, :]
    Cg = (cc * gamma)[None, :]
    beta = beta_ref[:][None, :]

    xr = x_ref[0, 0, :].astype(jnp.float32)
    yr = y_ref[0, 0, :].astype(jnp.float32)
    xn = xr * (1.0 / _X_SIZE) - 0.5
    yn = yr * (1.0 / _Y_SIZE) - 0.5
    var = (xn * xn * A2 + yn * yn * B2 + C2
           + 2.0 * (xn * yn * AB + xn * AC + yn * BC))
    s = jax.lax.rsqrt(var + 1e-5)
    u = (xn * s).reshape(_TOK, 1)
    v = (yn * s).reshape(_TOK, 1)
    s2 = s.reshape(_TOK, 1)

    hr = jnp.maximum(u * Ag + v * Bg + s2 * Cg + beta, 0.0)
    out = jnp.dot(hr.astype(jnp.bfloat16), w2_ref[:],
                  preferred_element_type=jnp.float32)
    out_ref[:, :] = out + b2_ref[:][None, :]


def kernel(x, y, W1, b1, gamma, beta, W2, b2):
    B, L = x.shape
    n = B * L
    grid = n // _TOK
    x3 = x.reshape(grid, 1, _TOK)
    y3 = y.reshape(grid, 1, _TOK)
    out = pl.pallas_call(
        _mlp_block,
        grid=(grid,),
        in_specs=[
            pl.BlockSpec((1, 1, _TOK), lambda i: (i, 0, 0)),
            pl.BlockSpec((1, 1, _TOK), lambda i: (i, 0, 0)),
            pl.BlockSpec((2, _D), lambda i: (0, 0)),
            pl.BlockSpec((_D,), lambda i: (0,)),
            pl.BlockSpec((_D,), lambda i: (0,)),
            pl.BlockSpec((_D,), lambda i: (0,)),
            pl.BlockSpec((_D, _E), lambda i: (0, 0)),
            pl.BlockSpec((_E,), lambda i: (0,)),
        ],
        out_specs=pl.BlockSpec((_TOK, _E), lambda i: (i, 0)),
        out_shape=jax.ShapeDtypeStruct((n, _E), jnp.float32),
        compiler_params=pltpu.CompilerParams(
            dimension_semantics=("parallel",)),
    )(x3, y3, W1, b1, gamma, beta, W2.astype(jnp.bfloat16), b2)
    return out.reshape(B, L, _E)


# speedup vs baseline: 1.1055x; 1.0454x over previous
import jax
import jax.numpy as jnp
from jax.experimental import pallas as pl
from jax.experimental.pallas import tpu as pltpu

_X_SIZE = 512.0
_Y_SIZE = 512.0
_D = 256
_E = 768
_TOK = 4096


def _mlp_block(x_ref, y_ref, w1_ref, b1_ref, gamma_ref, beta_ref, w2_ref,
               b2_ref, out_ref):
    w1 = w1_ref[:]
    a = w1[0]
    b = w1[1]
    c = b1_ref[:]
    abar = jnp.mean(a)
    bbar = jnp.mean(b)
    cbar = jnp.mean(c)
    ac = a - abar
    bc = b - bbar
    cc = c - cbar
    A2 = jnp.mean(ac * ac)
    B2 = jnp.mean(bc * bc)
    C2 = jnp.mean(cc * cc)
    AB = jnp.mean(ac * bc)
    AC = jnp.mean(ac * cc)
    BC = jnp.mean(bc * cc)
    gamma = gamma_ref[:]
    Ag = (ac * gamma)[None, :]
    Bg = (bc * gamma)[None, :]
    Cg = (cc * gamma)[None, :]
    beta = beta_ref[:][None, :]

    xr = x_ref[0, 0, :].astype(jnp.float32)
    yr = y_ref[0, 0, :].astype(jnp.float32)
    xn = xr * (1.0 / _X_SIZE) - 0.5
    yn = yr * (1.0 / _Y_SIZE) - 0.5
    var = (xn * xn * A2 + yn * yn * B2 + C2
           + 2.0 * (xn * yn * AB + xn * AC + yn * BC))
    s = jax.lax.rsqrt(var + 1e-5)
    u = (xn * s).reshape(_TOK, 1)
    v = (yn * s).reshape(_TOK, 1)
    s2 = s.reshape(_TOK, 1)

    hr = jnp.maximum(u * Ag + v * Bg + s2 * Cg + beta, 0.0)
    out = jnp.dot(hr.astype(jnp.bfloat16), w2_ref[:].astype(jnp.bfloat16),
                  preferred_element_type=jnp.float32)
    out_ref[:, :] = out + b2_ref[:][None, :]


def kernel(x, y, W1, b1, gamma, beta, W2, b2):
    B, L = x.shape
    n = B * L
    grid = n // _TOK
    x3 = x.reshape(grid, 1, _TOK)
    y3 = y.reshape(grid, 1, _TOK)
    out = pl.pallas_call(
        _mlp_block,
        grid=(grid,),
        in_specs=[
            pl.BlockSpec((1, 1, _TOK), lambda i: (i, 0, 0)),
            pl.BlockSpec((1, 1, _TOK), lambda i: (i, 0, 0)),
            pl.BlockSpec((2, _D), lambda i: (0, 0)),
            pl.BlockSpec((_D,), lambda i: (0,)),
            pl.BlockSpec((_D,), lambda i: (0,)),
            pl.BlockSpec((_D,), lambda i: (0,)),
            pl.BlockSpec((_D, _E), lambda i: (0, 0)),
            pl.BlockSpec((_E,), lambda i: (0,)),
        ],
        out_specs=pl.BlockSpec((_TOK, _E), lambda i: (i, 0)),
        out_shape=jax.ShapeDtypeStruct((n, _E), jnp.float32),
        compiler_params=pltpu.CompilerParams(
            dimension_semantics=("parallel",)),
    )(x3, y3, W1, b1, gamma, beta, W2, b2)
    return out.reshape(B, L, _E)
